# baseline (speedup 1.0000x reference)
import jax
import jax.numpy as jnp
from jax import lax
from jax.experimental import pallas as pl
from jax.experimental.pallas import tpu as pltpu
from jax.experimental.pallas import tpu_sc as plsc

VOCAB = 1000000
LSEQ = 200
D = 64
BATCH = 4096

NC = 2
NS = 16
NW = NC * NS

BB = BATCH // NW
WTILES = BATCH // 128
NBUF = 2


def _sc_body(tok_hbm, xh_hbm, xp_hbm, pos_hbm, out_hbm,
             xh0, xh1, xp0, xp1, g0, g1, o0, o1, pos_v,
             gsem0, gsem1, ssem0, ssem1):
    xh_v = (xh0, xh1)
    xp_v = (xp0, xp1)
    gbuf = (g0, g1)
    obuf = (o0, o1)
    gsem = (gsem0, gsem1)
    ssem = (ssem0, ssem1)

    w = lax.axis_index("s") * NC + lax.axis_index("c")

    pltpu.sync_copy(pos_hbm, pos_v)

    def fetch(b, l):
        row = l * WTILES + w
        pltpu.sync_copy(xh_hbm.at[row], xh_v[b])
        pltpu.sync_copy(xp_hbm.at[row], xp_v[b])
        pltpu.async_copy(tok_hbm.at[xh_v[b]], gbuf[b], gsem[b])

    for b in range(NBUF):
        fetch(b, b)

    iota = lax.iota(jnp.int32, 16)

    @pl.loop(0, LSEQ // NBUF)
    def _grp(t):
        for b in range(NBUF):
            l = t * NBUF + b
            pltpu.make_async_copy(tok_hbm.at[pl.ds(0, 128)], gbuf[b],
                                  gsem[b]).wait()

            @pl.when(t > 0)
            def _():
                pltpu.make_async_copy(
                    obuf[b], out_hbm.at[0, :, 0], ssem[b]).wait()

            pr = l // 2
            pc = (l % 2) * 64
            par = []
            for jg in range(8):
                par.append(xp_v[b][pl.ds(jg * 16, 16)])

            @pl.loop(0, 8)
            def _g(g):
                for s in range(8):
                    f = g * 8 + s
                    ps = plsc.load_gather(
                        pos_v, [jnp.full((16,), pr, jnp.int32),
                                jnp.full((16,), pc + f, jnp.int32)])
                    for jg in range(8):
                        ridx = iota + (jg * 16)
                        cidx = par[jg] + f
                        val = plsc.load_gather(gbuf[b], [ridx, cidx]) + ps
                        obuf[b][g, s, pl.ds(jg * 16, 16)] = val

            pltpu.async_copy(obuf[b], out_hbm.at[l, :, w], ssem[b])

            @pl.when(l + NBUF < LSEQ)
            def _():
                fetch(b, l + NBUF)

    for b in range(NBUF):
        pltpu.make_async_copy(obuf[b], out_hbm.at[0, :, 0], ssem[b]).wait()


@jax.jit
def _sc_embed(tok2, xh2, xp2, pos2):
    mesh = plsc.VectorSubcoreMesh(core_axis_name="c", subcore_axis_name="s")
    fn = pl.kernel(
        _sc_body,
        out_type=jax.ShapeDtypeStruct((LSEQ, 8, WTILES, 8, 128), jnp.float32),
        mesh=mesh,
        scratch_types=[
            pltpu.VMEM((128,), jnp.int32),
            pltpu.VMEM((128,), jnp.int32),
            pltpu.VMEM((128,), jnp.int32),
            pltpu.VMEM((128,), jnp.int32),
            pltpu.VMEM((128, 128), jnp.float32),
            pltpu.VMEM((128, 128), jnp.float32),
            pltpu.VMEM((8, 8, 128), jnp.float32),
            pltpu.VMEM((8, 8, 128), jnp.float32),
            pltpu.VMEM((100, 128), jnp.float32),
            pltpu.SemaphoreType.DMA,
            pltpu.SemaphoreType.DMA,
            pltpu.SemaphoreType.DMA,
            pltpu.SemaphoreType.DMA,
        ],
        compiler_params=pltpu.CompilerParams(use_tc_tiling_on_sc=False,
                                             needs_layout_passes=False),
    )
    return fn(tok2, xh2, xp2, pos2)


def kernel(x, token_table, pos_table):
    xi = x.astype(jnp.int32)
    tok2 = token_table.reshape(VOCAB // 2, 128)
    xh2 = (xi >> 1).T.reshape(LSEQ * WTILES, 128)
    xp2 = ((xi & 1) * 64).T.reshape(LSEQ * WTILES, 128)
    pos2 = pos_table.reshape(100, 128)
    out5 = _sc_embed(tok2, xh2, xp2, pos2)
    return out5.transpose(2, 4, 0, 1, 3).reshape(BATCH, LSEQ, D)


# speedup vs baseline: 1.0930x; 1.0930x over previous
import jax
import jax.numpy as jnp
from jax import lax
from jax.experimental import pallas as pl
from jax.experimental.pallas import tpu as pltpu
from jax.experimental.pallas import tpu_sc as plsc

VOCAB = 1000000
LSEQ = 200
D = 64
BATCH = 4096

NC = 2
NS = 16
NW = NC * NS

WTILES = BATCH // 128
NBUF = 2


def _sc_body(tok_hbm, xi_hbm, pos_hbm, out_hbm,
             xall, g0, g1, o0, o1, pos_v,
             gsem0, gsem1, ssem0, ssem1):
    gbuf = (g0, g1)
    obuf = (o0, o1)
    gsem = (gsem0, gsem1)
    ssem = (ssem0, ssem1)

    w = lax.axis_index("s") * NC + lax.axis_index("c")

    pltpu.sync_copy(pos_hbm, pos_v)
    pltpu.sync_copy(xi_hbm.at[:, w], xall)

    def fetch(b, l):
        pltpu.async_copy(tok_hbm.at[xall.at[l]], gbuf[b], gsem[b])

    for b in range(NBUF):
        fetch(b, b)

    iota = lax.iota(jnp.int32, 16)

    @pl.loop(0, LSEQ // NBUF)
    def _grp(t):
        for b in range(NBUF):
            l = t * NBUF + b
            pltpu.make_async_copy(tok_hbm.at[pl.ds(0, 128)], gbuf[b],
                                  gsem[b]).wait()

            @pl.when(t > 0)
            def _():
                pltpu.make_async_copy(
                    obuf[b], out_hbm.at[0, :, 0], ssem[b]).wait()

            pr = l // 2
            pc = (l % 2) * 64

            @pl.loop(0, 8)
            def _g(g):
                for s in range(8):
                    f = g * 8 + s
                    fv = jnp.full((16,), f, jnp.int32)
                    ps = plsc.load_gather(
                        pos_v, [jnp.full((16,), pr, jnp.int32),
                                jnp.full((16,), pc, jnp.int32) + fv])
                    for jg in range(8):
                        val = plsc.load_gather(
                            gbuf[b], [iota + (jg * 16), fv]) + ps
                        obuf[b][g, s, pl.ds(jg * 16, 16)] = val

            pltpu.async_copy(obuf[b], out_hbm.at[l, :, w], ssem[b])

            @pl.when(l + NBUF < LSEQ)
            def _():
                fetch(b, l + NBUF)

    for b in range(NBUF):
        pltpu.make_async_copy(obuf[b], out_hbm.at[0, :, 0], ssem[b]).wait()


@jax.jit
def _sc_embed(tok, xi3, pos2):
    mesh = plsc.VectorSubcoreMesh(core_axis_name="c", subcore_axis_name="s")
    fn = pl.kernel(
        _sc_body,
        out_type=jax.ShapeDtypeStruct((LSEQ, 8, WTILES, 8, 128), jnp.float32),
        mesh=mesh,
        scratch_types=[
            pltpu.VMEM((LSEQ, 128), jnp.int32),
            pltpu.VMEM((128, D), jnp.float32),
            pltpu.VMEM((128, D), jnp.float32),
            pltpu.VMEM((8, 8, 128), jnp.float32),
            pltpu.VMEM((8, 8, 128), jnp.float32),
            pltpu.VMEM((100, 128), jnp.float32),
            pltpu.SemaphoreType.DMA,
            pltpu.SemaphoreType.DMA,
            pltpu.SemaphoreType.DMA,
            pltpu.SemaphoreType.DMA,
        ],
        compiler_params=pltpu.CompilerParams(use_tc_tiling_on_sc=False,
                                             needs_layout_passes=False),
    )
    return fn(tok, xi3, pos2)


def kernel(x, token_table, pos_table):
    xi3 = x.astype(jnp.int32).T.reshape(LSEQ, WTILES, 128)
    pos2 = pos_table.reshape(100, 128)
    out5 = _sc_embed(token_table, xi3, pos2)
    return out5.transpose(2, 4, 0, 1, 3).reshape(BATCH, LSEQ, D)


# speedup vs baseline: 1.3146x; 1.2027x over previous
import jax
import jax.numpy as jnp
from jax import lax
from jax.experimental import pallas as pl
from jax.experimental.pallas import tpu as pltpu
from jax.experimental.pallas import tpu_sc as plsc

VOCAB = 1000000
LSEQ = 200
D = 64
BATCH = 4096

NC = 2
NS = 16
NW = NC * NS

WTILES = BATCH // 128
NBUF = 2


def _sc_body(tok_hbm, xi_hbm, pos_hbm, out_hbm,
             xall, g0, g1, o0, o1, pos_v,
             gsem0, gsem1, ssem0, ssem1):
    gbuf = (g0, g1)
    obuf = (o0, o1)
    gsem = (gsem0, gsem1)
    ssem = (ssem0, ssem1)

    w = lax.axis_index("s") * NC + lax.axis_index("c")

    pltpu.sync_copy(pos_hbm, pos_v)
    pltpu.sync_copy(xi_hbm.at[:, w], xall)

    def fetch(b, l):
        pltpu.async_copy(tok_hbm.at[xall.at[l]], gbuf[b], gsem[b])

    for b in range(NBUF):
        fetch(b, b)

    iota = lax.iota(jnp.int32, 16)

    @pl.loop(0, LSEQ // NBUF)
    def _grp(t):
        for b in range(NBUF):
            l = t * NBUF + b
            pltpu.make_async_copy(tok_hbm.at[pl.ds(0, 128)], gbuf[b],
                                  gsem[b]).wait()

            @pl.when(t > 0)
            def _():
                pltpu.make_async_copy(
                    obuf[b], out_hbm.at[0, :, 0], ssem[b]).wait()

            pr = l // 2
            pc = (l % 2) * 64

            @pl.loop(0, 8)
            def _g(g):
                for s in range(8):
                    f = g * 8 + s
                    fv = jnp.full((16,), f, jnp.int32)
                    ps = plsc.load_gather(
                        pos_v, [jnp.full((16,), pr, jnp.int32),
                                jnp.full((16,), pc, jnp.int32) + fv])
                    vals = [plsc.load_gather(gbuf[b], [iota + (jg * 16), fv])
                            for jg in range(8)]
                    for jg in range(8):
                        obuf[b][g, s, pl.ds(jg * 16, 16)] = vals[jg] + ps

            pltpu.async_copy(obuf[b], out_hbm.at[l, :, w], ssem[b])

            @pl.when(l + NBUF < LSEQ)
            def _():
                fetch(b, l + NBUF)

    for b in range(NBUF):
        pltpu.make_async_copy(obuf[b], out_hbm.at[0, :, 0], ssem[b]).wait()


@jax.jit
def _sc_embed(tok, xi3, pos2):
    mesh = plsc.VectorSubcoreMesh(core_axis_name="c", subcore_axis_name="s")
    fn = pl.kernel(
        _sc_body,
        out_type=jax.ShapeDtypeStruct((LSEQ, 8, WTILES, 8, 128), jnp.float32),
        mesh=mesh,
        scratch_types=[
            pltpu.VMEM((LSEQ, 128), jnp.int32),
            pltpu.VMEM((128, D), jnp.float32),
            pltpu.VMEM((128, D), jnp.float32),
            pltpu.VMEM((8, 8, 128), jnp.float32),
            pltpu.VMEM((8, 8, 128), jnp.float32),
            pltpu.VMEM((100, 128), jnp.float32),
            pltpu.SemaphoreType.DMA,
            pltpu.SemaphoreType.DMA,
            pltpu.SemaphoreType.DMA,
            pltpu.SemaphoreType.DMA,
        ],
        compiler_params=pltpu.CompilerParams(use_tc_tiling_on_sc=False,
                                             needs_layout_passes=False),
    )
    return fn(tok, xi3, pos2)


def kernel(x, token_table, pos_table):
    xi3 = x.astype(jnp.int32).T.reshape(LSEQ, WTILES, 128)
    pos2 = pos_table.reshape(100, 128)
    out5 = _sc_embed(token_table, xi3, pos2)
    return out5.transpose(2, 4, 0, 1, 3).reshape(BATCH, LSEQ, D)
